# initial kernel scaffold (unmeasured)
import jax
import jax.numpy as jnp
from jax import lax
from jax.experimental import pallas as pl
from jax.experimental.pallas import tpu as pltpu

N_DEV = 4
M_PER = 1024
N_COLS = 8192
B = 8
WN = N_COLS // B


def kernel(x, w_mat):
    x = x.astype(jnp.bfloat16)
    w = w_mat.astype(jnp.bfloat16)

    def body(x_ref, w_ref, out_ref, comm_ref, send_sems, recv_sems):
        my = lax.axis_index("i")
        left = lax.rem(my + N_DEV - 1, N_DEV)
        right = lax.rem(my + 1, N_DEV)

        barrier_sem = pltpu.get_barrier_semaphore()
        for nbr in (left, right):
            pl.semaphore_signal(
                barrier_sem, inc=1,
                device_id=(nbr,), device_id_type=pl.DeviceIdType.MESH,
            )
        pl.semaphore_wait(barrier_sem, 2)

        hop = 0
        for b in range(B):
            cols = pl.ds(b * WN, WN)
            c0 = lax.rem(my + N_DEV - 1, N_DEV)
            comm_ref[hop % 2] = jnp.dot(
                x_ref[pl.ds(c0 * M_PER, M_PER), :], w_ref[:, cols],
                preferred_element_type=jnp.float32,
            ).astype(jnp.bfloat16)
            for s in range(N_DEV - 1):
                send_slot = hop % 2
                recv_slot = (hop + 1) % 2
                rdma = pltpu.make_async_remote_copy(
                    src_ref=comm_ref.at[send_slot],
                    dst_ref=comm_ref.at[recv_slot],
                    send_sem=send_sems.at[send_slot],
                    recv_sem=recv_sems.at[recv_slot],
                    device_id=(right,),
                    device_id_type=pl.DeviceIdType.MESH,
                )
                rdma.start()
                rdma.wait()
                c = lax.rem(my + 2 * N_DEV - 2 - s, N_DEV)
                local = jnp.dot(
                    x_ref[pl.ds(c * M_PER, M_PER), :], w_ref[:, cols],
                    preferred_element_type=jnp.float32,
                )
                acc = local + comm_ref[recv_slot].astype(jnp.float32)
                if s < N_DEV - 2:
                    comm_ref[recv_slot] = acc.astype(jnp.bfloat16)
                else:
                    out_ref[:, cols] = jnp.maximum(acc, 0.0).astype(
                        jnp.bfloat16
                    )
                hop += 1

    return pl.pallas_call(
        body,
        out_shape=jax.ShapeDtypeStruct((M_PER, N_COLS), jnp.bfloat16),
        in_specs=[
            pl.BlockSpec(memory_space=pltpu.VMEM),
            pl.BlockSpec(memory_space=pltpu.VMEM),
        ],
        out_specs=pl.BlockSpec(memory_space=pltpu.VMEM),
        scratch_shapes=[
            pltpu.VMEM((2, M_PER, WN), jnp.bfloat16),
            pltpu.SemaphoreType.DMA((2,)),
            pltpu.SemaphoreType.DMA((2,)),
        ],
        compiler_params=pltpu.CompilerParams(collective_id=0),
    )(x, w)


# baseline (device time: 718495 ns/iter reference)
import jax
import jax.numpy as jnp
from jax import lax
from jax.experimental import pallas as pl
from jax.experimental.pallas import tpu as pltpu

N_DEV = 4
M_PER = 1024
N_COLS = 8192
B = 8
WN = N_COLS // B


def kernel(x, w_mat):
    x = x.astype(jnp.bfloat16)
    w = w_mat.astype(jnp.bfloat16)

    def body(x_ref, w_ref, out_ref, comm_ref, send_sems, recv_sems):
        my = lax.axis_index("i")
        left = lax.rem(my + N_DEV - 1, N_DEV)
        right = lax.rem(my + 1, N_DEV)

        barrier_sem = pltpu.get_barrier_semaphore()
        for nbr in (left, right):
            pl.semaphore_signal(
                barrier_sem, inc=1,
                device_id=(nbr,), device_id_type=pl.DeviceIdType.MESH,
            )
        pl.semaphore_wait(barrier_sem, 2)

        hop = 0
        for b in range(B):
            cols = pl.ds(b * WN, WN)
            c0 = lax.rem(my + N_DEV - 1, N_DEV)
            comm_ref[hop % 2] = jnp.dot(
                x_ref[pl.ds(c0 * M_PER, M_PER), :], w_ref[:, cols],
                preferred_element_type=jnp.float32,
            ).astype(jnp.bfloat16)
            for s in range(N_DEV - 1):
                send_slot = hop % 2
                recv_slot = (hop + 1) % 2
                rdma = pltpu.make_async_remote_copy(
                    src_ref=comm_ref.at[send_slot],
                    dst_ref=comm_ref.at[recv_slot],
                    send_sem=send_sems.at[send_slot],
                    recv_sem=recv_sems.at[recv_slot],
                    device_id=(right,),
                    device_id_type=pl.DeviceIdType.MESH,
                )
                rdma.start()
                rdma.wait()
                c = lax.rem(my + 2 * N_DEV - 2 - s, N_DEV)
                local = jnp.dot(
                    x_ref[pl.ds(c * M_PER, M_PER), :], w_ref[:, cols],
                    preferred_element_type=jnp.float32,
                )
                acc = local + comm_ref[recv_slot].astype(jnp.float32)
                if s < N_DEV - 2:
                    comm_ref[recv_slot] = acc.astype(jnp.bfloat16)
                else:
                    out_ref[:, cols] = jnp.maximum(acc, 0.0).astype(
                        jnp.bfloat16
                    )
                hop += 1

    return pl.pallas_call(
        body,
        out_shape=jax.ShapeDtypeStruct((M_PER, N_COLS), jnp.bfloat16),
        in_specs=[
            pl.BlockSpec(memory_space=pltpu.VMEM),
            pl.BlockSpec(memory_space=pltpu.VMEM),
        ],
        out_specs=pl.BlockSpec(memory_space=pltpu.VMEM),
        scratch_shapes=[
            pltpu.VMEM((2, M_PER, WN), jnp.bfloat16),
            pltpu.SemaphoreType.DMA((2,)),
            pltpu.SemaphoreType.DMA((2,)),
        ],
        compiler_params=pltpu.CompilerParams(
            collective_id=0,
            vmem_limit_bytes=64 * 1024 * 1024,
        ),
    )(x, w)


# device time: 411329 ns/iter; 1.7468x vs baseline; 1.7468x over previous
import jax
import jax.numpy as jnp
from jax import lax
from jax.experimental import pallas as pl
from jax.experimental.pallas import tpu as pltpu

N_DEV = 4
M_PER = 1024
N_COLS = 8192
B = 4
WN = N_COLS // B
HW = WN // 2


def kernel(x, w_mat):
    x = x.astype(jnp.bfloat16)
    w = w_mat.astype(jnp.bfloat16)

    def body(x_ref, w_ref, out_ref, commR, commL, sR, rR, sL, rL):
        my = lax.axis_index("i")
        left = lax.rem(my + N_DEV - 1, N_DEV)
        right = lax.rem(my + 1, N_DEV)

        def mm(row, col, width):
            return jnp.dot(
                x_ref[pl.ds(row * M_PER, M_PER), :],
                w_ref[:, pl.ds(col, width)],
                preferred_element_type=jnp.float32,
            )

        rowFR = lax.rem(my + N_DEV - 1, N_DEV)
        rowFL = lax.rem(my + 1, N_DEV)

        barrier_sem = pltpu.get_barrier_semaphore()
        for nbr in (left, right):
            pl.semaphore_signal(
                barrier_sem, inc=1,
                device_id=(nbr,), device_id_type=pl.DeviceIdType.MESH,
            )
        commR[0] = mm(rowFR, 0, HW).astype(jnp.bfloat16)
        commL[0] = mm(rowFL, HW, HW).astype(jnp.bfloat16)
        pl.semaphore_wait(barrier_sem, 2)

        h = 0
        for b in range(B):
            colR = b * WN
            colL = b * WN + HW
            freshR = []
            freshL = []
            for s in range(N_DEV - 1):
                ss = h % 2
                rs = (h + 1) % 2
                rdmaR = pltpu.make_async_remote_copy(
                    src_ref=commR.at[ss], dst_ref=commR.at[rs],
                    send_sem=sR.at[ss], recv_sem=rR.at[rs],
                    device_id=(right,), device_id_type=pl.DeviceIdType.MESH,
                )
                rdmaL = pltpu.make_async_remote_copy(
                    src_ref=commL.at[ss], dst_ref=commL.at[rs],
                    send_sem=sL.at[ss], recv_sem=rL.at[rs],
                    device_id=(left,), device_id_type=pl.DeviceIdType.MESH,
                )
                rdmaR.start()
                rdmaL.start()
                addR = mm(lax.rem(my + 2 * N_DEV - 2 - s, N_DEV), colR, HW)
                addL = mm(lax.rem(my + 2 + s, N_DEV), colL, HW)
                if b + 1 < B and s >= 1:
                    off = (s - 1) * (HW // 2)
                    freshR.append(
                        mm(rowFR, (b + 1) * WN + off, HW // 2)
                        .astype(jnp.bfloat16)
                    )
                    freshL.append(
                        mm(rowFL, (b + 1) * WN + HW + off, HW // 2)
                        .astype(jnp.bfloat16)
                    )
                rdmaR.wait()
                rdmaL.wait()
                accR = addR + commR[rs].astype(jnp.float32)
                accL = addL + commL[rs].astype(jnp.float32)
                if s < N_DEV - 2:
                    commR[rs] = accR.astype(jnp.bfloat16)
                    commL[rs] = accL.astype(jnp.bfloat16)
                else:
                    out_ref[:, pl.ds(colR, HW)] = jnp.maximum(
                        accR, 0.0
                    ).astype(jnp.bfloat16)
                    out_ref[:, pl.ds(colL, HW)] = jnp.maximum(
                        accL, 0.0
                    ).astype(jnp.bfloat16)
                    if b + 1 < B:
                        commR[rs, :, : HW // 2] = freshR[0]
                        commR[rs, :, HW // 2 :] = freshR[1]
                        commL[rs, :, : HW // 2] = freshL[0]
                        commL[rs, :, HW // 2 :] = freshL[1]
                h += 1

    return pl.pallas_call(
        body,
        out_shape=jax.ShapeDtypeStruct((M_PER, N_COLS), jnp.bfloat16),
        in_specs=[
            pl.BlockSpec(memory_space=pltpu.VMEM),
            pl.BlockSpec(memory_space=pltpu.VMEM),
        ],
        out_specs=pl.BlockSpec(memory_space=pltpu.VMEM),
        scratch_shapes=[
            pltpu.VMEM((2, M_PER, HW), jnp.bfloat16),
            pltpu.VMEM((2, M_PER, HW), jnp.bfloat16),
            pltpu.SemaphoreType.DMA((2,)),
            pltpu.SemaphoreType.DMA((2,)),
            pltpu.SemaphoreType.DMA((2,)),
            pltpu.SemaphoreType.DMA((2,)),
        ],
        compiler_params=pltpu.CompilerParams(
            collective_id=0,
            vmem_limit_bytes=64 * 1024 * 1024,
        ),
    )(x, w)


# device time: 328942 ns/iter; 2.1843x vs baseline; 1.2505x over previous
import jax
import jax.numpy as jnp
from jax import lax
from jax.experimental import pallas as pl
from jax.experimental.pallas import tpu as pltpu

N_DEV = 4
M_PER = 1024
N_COLS = 8192
B = 4
WN = N_COLS // B
HW = WN // 2
SC = HW // 2
N_FLIGHT = B * (N_DEV - 1)


def kernel(x, w_mat):
    x = x.astype(jnp.bfloat16)
    w = w_mat.astype(jnp.bfloat16)

    def body(x_ref, w_ref, out_ref,
             comm0, comm1, comm2, comm3, out_stage,
             ss0, rs0, ss1, rs1, ss2, rs2, ss3, rs3, out_cp_sems):
        my = lax.axis_index("i")
        left = lax.rem(my + N_DEV - 1, N_DEV)
        right = lax.rem(my + 1, N_DEV)

        comm = [comm0, comm1, comm2, comm3]
        send_sems = [ss0, ss1, ss2, ss3]
        recv_sems = [rs0, rs1, rs2, rs3]
        dev_of = [right, right, left, left]

        def mm(row, col, width):
            return jnp.dot(
                x_ref[pl.ds(row * M_PER, M_PER), :],
                w_ref[:, pl.ds(col, width)],
                preferred_element_type=jnp.float32,
            ).astype(jnp.bfloat16)

        def col_of(st, b):
            return b * WN + st * SC

        rowFR = lax.rem(my + N_DEV - 1, N_DEV)
        rowFL = lax.rem(my + 1, N_DEV)
        row_fresh = [rowFR, rowFR, rowFL, rowFL]

        def make_rdma(st, h):
            s_slot = h % 2
            r_slot = (h + 1) % 2
            return pltpu.make_async_remote_copy(
                src_ref=comm[st].at[s_slot],
                dst_ref=comm[st].at[r_slot],
                send_sem=send_sems[st].at[s_slot],
                recv_sem=recv_sems[st].at[r_slot],
                device_id=(dev_of[st],),
                device_id_type=pl.DeviceIdType.MESH,
            )

        barrier_sem = pltpu.get_barrier_semaphore()
        for nbr in (left, right):
            pl.semaphore_signal(
                barrier_sem, inc=1,
                device_id=(nbr,), device_id_type=pl.DeviceIdType.MESH,
            )

        def out_cp(b):
            return pltpu.make_async_copy(
                out_stage.at[b % 2],
                out_ref.at[:, pl.ds(b * WN, WN)],
                out_cp_sems.at[b % 2],
            )

        rdmas = {}

        def start_flight(st, h):
            r = make_rdma(st, h)
            rdmas[(st, h)] = r
            r.start()

        comm[0][0] = mm(row_fresh[0], col_of(0, 0), SC)
        pl.semaphore_wait(barrier_sem, 2)
        start_flight(0, 0)
        for st in (1, 2, 3):
            comm[st][0] = mm(row_fresh[st], col_of(st, 0), SC)
            start_flight(st, 0)

        for h in range(N_FLIGHT):
            b, s = divmod(h, N_DEV - 1)
            r_slot = (h + 1) % 2
            addR = mm(lax.rem(my + 2 * N_DEV - 2 - s, N_DEV), b * WN, HW)
            addL = mm(lax.rem(my + 2 + s, N_DEV), b * WN + HW, HW)
            adds = [addR[:, :SC], addR[:, SC:], addL[:, :SC], addL[:, SC:]]
            if b + 1 < B and s >= 1:
                off = (s - 1) * SC
                fr = mm(rowFR, (b + 1) * WN + off, SC)
                fl = mm(rowFL, (b + 1) * WN + HW + off, SC)
                if s == 1:
                    fresh = {0: fr, 2: fl}
                else:
                    fresh[1] = fr
                    fresh[3] = fl
            relu_out = {}
            for st in range(4):
                rdmas[(st, h)].wait()
                acc = adds[st] + comm[st][r_slot]
                if s < N_DEV - 2:
                    comm[st][r_slot] = acc
                else:
                    if b + 1 < B:
                        comm[st][r_slot] = fresh[st]
                    relu_out[st] = jnp.maximum(acc, 0.0)
                if h + 1 < N_FLIGHT:
                    start_flight(st, h + 1)
            if s == N_DEV - 2:
                p = b % 2
                if b >= 2:
                    out_cp(b - 2).wait()
                for st in range(4):
                    out_stage[p, :, pl.ds(st * SC, SC)] = relu_out[st]
                out_cp(b).start()

        for b in (B - 2, B - 1):
            out_cp(b).wait()

    return pl.pallas_call(
        body,
        out_shape=jax.ShapeDtypeStruct((M_PER, N_COLS), jnp.bfloat16),
        in_specs=[
            pl.BlockSpec(memory_space=pltpu.VMEM),
            pl.BlockSpec(memory_space=pltpu.VMEM),
        ],
        out_specs=pl.BlockSpec(memory_space=pl.ANY),
        scratch_shapes=[
            pltpu.VMEM((2, M_PER, SC), jnp.bfloat16),
            pltpu.VMEM((2, M_PER, SC), jnp.bfloat16),
            pltpu.VMEM((2, M_PER, SC), jnp.bfloat16),
            pltpu.VMEM((2, M_PER, SC), jnp.bfloat16),
            pltpu.VMEM((2, M_PER, WN), jnp.bfloat16),
            pltpu.SemaphoreType.DMA((2,)),
            pltpu.SemaphoreType.DMA((2,)),
            pltpu.SemaphoreType.DMA((2,)),
            pltpu.SemaphoreType.DMA((2,)),
            pltpu.SemaphoreType.DMA((2,)),
            pltpu.SemaphoreType.DMA((2,)),
            pltpu.SemaphoreType.DMA((2,)),
            pltpu.SemaphoreType.DMA((2,)),
            pltpu.SemaphoreType.DMA((2,)),
        ],
        compiler_params=pltpu.CompilerParams(
            collective_id=0,
            vmem_limit_bytes=64 * 1024 * 1024,
        ),
    )(x, w)


# device time: 310548 ns/iter; 2.3136x vs baseline; 1.0592x over previous
import jax
import jax.numpy as jnp
from jax import lax
from jax.experimental import pallas as pl
from jax.experimental.pallas import tpu as pltpu

N_DEV = 4
M_PER = 1024
N_COLS = 8192
B = 4
WN = N_COLS // B
HW = WN // 2
SC = HW // 2
N_FLIGHT = B * (N_DEV - 1)


def kernel(x, w_mat):
    x = x.astype(jnp.bfloat16)

    def body(x_ref, w_ref, out_ref,
             comm0, comm1, comm2, comm3, out_stage, w_stage, w_bf,
             ss0, rs0, ss1, rs1, ss2, rs2, ss3, rs3,
             out_cp_sems, w_sems):
        my = lax.axis_index("i")
        left = lax.rem(my + N_DEV - 1, N_DEV)
        right = lax.rem(my + 1, N_DEV)

        comm = [comm0, comm1, comm2, comm3]
        send_sems = [ss0, ss1, ss2, ss3]
        recv_sems = [rs0, rs1, rs2, rs3]
        dev_of = [right, right, left, left]

        def mm(row, col, width):
            return jnp.dot(
                x_ref[pl.ds(row * M_PER, M_PER), :],
                w_bf[:, pl.ds(col, width)],
                preferred_element_type=jnp.float32,
            ).astype(jnp.bfloat16)

        def w_dma(b):
            return pltpu.make_async_copy(
                w_ref.at[:, pl.ds(b * WN, WN)],
                w_stage,
                w_sems.at[b % 2],
            )

        def w_cast(b):
            w_bf[:, pl.ds(b * WN, WN)] = w_stage[:, :].astype(jnp.bfloat16)

        def col_of(st, b):
            return b * WN + st * SC

        rowFR = lax.rem(my + N_DEV - 1, N_DEV)
        rowFL = lax.rem(my + 1, N_DEV)
        row_fresh = [rowFR, rowFR, rowFL, rowFL]

        def make_rdma(st, h):
            s_slot = h % 2
            r_slot = (h + 1) % 2
            return pltpu.make_async_remote_copy(
                src_ref=comm[st].at[s_slot],
                dst_ref=comm[st].at[r_slot],
                send_sem=send_sems[st].at[s_slot],
                recv_sem=recv_sems[st].at[r_slot],
                device_id=(dev_of[st],),
                device_id_type=pl.DeviceIdType.MESH,
            )

        barrier_sem = pltpu.get_barrier_semaphore()
        for nbr in (left, right):
            pl.semaphore_signal(
                barrier_sem, inc=1,
                device_id=(nbr,), device_id_type=pl.DeviceIdType.MESH,
            )

        def out_cp(b):
            return pltpu.make_async_copy(
                out_stage,
                out_ref.at[:, pl.ds(b * WN, WN)],
                out_cp_sems.at[b % 2],
            )

        rdmas = {}

        def start_flight(st, h):
            r = make_rdma(st, h)
            rdmas[(st, h)] = r
            r.start()

        w_dma(0).start()
        w_dma(0).wait()
        w_cast(0)
        w_dma(1).start()
        comm[0][0] = mm(row_fresh[0], col_of(0, 0), SC)
        pl.semaphore_wait(barrier_sem, 2)
        start_flight(0, 0)
        for st in (1, 2, 3):
            comm[st][0] = mm(row_fresh[st], col_of(st, 0), SC)
            start_flight(st, 0)

        for h in range(N_FLIGHT):
            b, s = divmod(h, N_DEV - 1)
            r_slot = (h + 1) % 2
            addR = mm(lax.rem(my + 2 * N_DEV - 2 - s, N_DEV), b * WN, HW)
            addL = mm(lax.rem(my + 2 + s, N_DEV), b * WN + HW, HW)
            adds = [addR[:, :SC], addR[:, SC:], addL[:, :SC], addL[:, SC:]]
            if s == 0 and b + 1 < B:
                w_dma(b + 1).wait()
                w_cast(b + 1)
                if b + 2 < B:
                    w_dma(b + 2).start()
            if b + 1 < B and s >= 1:
                off = (s - 1) * SC
                fr = mm(rowFR, (b + 1) * WN + off, SC)
                fl = mm(rowFL, (b + 1) * WN + HW + off, SC)
                if s == 1:
                    fresh = {0: fr, 2: fl}
                else:
                    fresh[1] = fr
                    fresh[3] = fl
            relu_out = {}
            for st in range(4):
                rdmas[(st, h)].wait()
                acc = adds[st] + comm[st][r_slot]
                if s < N_DEV - 2:
                    comm[st][r_slot] = acc
                else:
                    if b + 1 < B:
                        comm[st][r_slot] = fresh[st]
                    relu_out[st] = jnp.maximum(acc, 0.0)
                if h + 1 < N_FLIGHT:
                    start_flight(st, h + 1)
            if s == N_DEV - 2:
                if b >= 1:
                    out_cp(b - 1).wait()
                for st in range(4):
                    out_stage[:, pl.ds(st * SC, SC)] = relu_out[st]
                out_cp(b).start()

        out_cp(B - 1).wait()

    return pl.pallas_call(
        body,
        out_shape=jax.ShapeDtypeStruct((M_PER, N_COLS), jnp.bfloat16),
        in_specs=[
            pl.BlockSpec(memory_space=pltpu.VMEM),
            pl.BlockSpec(memory_space=pl.ANY),
        ],
        out_specs=pl.BlockSpec(memory_space=pl.ANY),
        scratch_shapes=[
            pltpu.VMEM((2, M_PER, SC), jnp.bfloat16),
            pltpu.VMEM((2, M_PER, SC), jnp.bfloat16),
            pltpu.VMEM((2, M_PER, SC), jnp.bfloat16),
            pltpu.VMEM((2, M_PER, SC), jnp.bfloat16),
            pltpu.VMEM((M_PER, WN), jnp.bfloat16),
            pltpu.VMEM((M_PER, WN), jnp.float32),
            pltpu.VMEM((M_PER, N_COLS), jnp.bfloat16),
            pltpu.SemaphoreType.DMA((2,)),
            pltpu.SemaphoreType.DMA((2,)),
            pltpu.SemaphoreType.DMA((2,)),
            pltpu.SemaphoreType.DMA((2,)),
            pltpu.SemaphoreType.DMA((2,)),
            pltpu.SemaphoreType.DMA((2,)),
            pltpu.SemaphoreType.DMA((2,)),
            pltpu.SemaphoreType.DMA((2,)),
            pltpu.SemaphoreType.DMA((2,)),
            pltpu.SemaphoreType.DMA((2,)),
        ],
        compiler_params=pltpu.CompilerParams(
            collective_id=0,
            vmem_limit_bytes=64 * 1024 * 1024,
        ),
    )(x, w_mat)
